# baseline (speedup 1.0000x reference)
import functools

import jax
import jax.numpy as jnp
from jax.experimental import pallas as pl
from jax.experimental.pallas import tpu as pltpu

_N = 1 << 24
_NA = 4
_CHUNK = _N // _NA
_GRID = 64
_BLK = _N // _GRID
_TPB = _GRID // _NA


def _segsum_softmax_body(x_ref, o_ref, acc_ref):
    i = pl.program_id(0)

    @pl.when(i == 0)
    def _init():
        acc_ref[...] = jnp.zeros_like(acc_ref)

    blk = x_ref[...].reshape(_BLK // 128, 128)
    partial = jnp.sum(blk, axis=0, keepdims=True)
    b = i // _TPB
    row = jax.lax.broadcasted_iota(jnp.int32, (_NA, 128), 0)
    acc_ref[...] += jnp.where(row == b, partial, 0.0)

    @pl.when(i == _GRID - 1)
    def _finish():
        s = jnp.sum(acc_ref[...], axis=1)
        m = jnp.max(s)
        e = jnp.exp(s - m)
        o_ref[...] = e / jnp.sum(e)


@jax.jit
def kernel(probs):
    return pl.pallas_call(
        _segsum_softmax_body,
        grid=(_GRID,),
        in_specs=[pl.BlockSpec((_BLK,), lambda i: (i,))],
        out_specs=pl.BlockSpec((_NA,), lambda i: (0,)),
        out_shape=jax.ShapeDtypeStruct((_NA,), jnp.float32),
        scratch_shapes=[pltpu.VMEM((_NA, 128), jnp.float32)],
        compiler_params=pltpu.CompilerParams(
            dimension_semantics=("arbitrary",),
        ),
    )(probs)


# speedup vs baseline: 1.8161x; 1.8161x over previous
import functools

import jax
import jax.numpy as jnp
from jax.experimental import pallas as pl
from jax.experimental.pallas import tpu as pltpu

_N = 1 << 24
_NA = 4
_CHUNK = _N // _NA
_GRID = 16
_BLK = _N // _GRID
_TPB = _GRID // _NA


def _segsum_softmax_body(x_ref, o_ref, acc_ref):
    i = pl.program_id(0)

    @pl.when(i == 0)
    def _init():
        acc_ref[...] = jnp.zeros_like(acc_ref)

    blk = x_ref[...].reshape(_BLK // 128, 128)
    partial = jnp.sum(blk, axis=0, keepdims=True)
    b = i // _TPB
    row = jax.lax.broadcasted_iota(jnp.int32, (_NA, 128), 0)
    acc_ref[...] += jnp.where(row == b, partial, 0.0)

    @pl.when(i == _GRID - 1)
    def _finish():
        s = jnp.sum(acc_ref[...], axis=1)
        m = jnp.max(s)
        e = jnp.exp(s - m)
        o_ref[...] = e / jnp.sum(e)


@jax.jit
def kernel(probs):
    return pl.pallas_call(
        _segsum_softmax_body,
        grid=(_GRID,),
        in_specs=[pl.BlockSpec((_BLK,), lambda i: (i,))],
        out_specs=pl.BlockSpec((_NA,), lambda i: (0,)),
        out_shape=jax.ShapeDtypeStruct((_NA,), jnp.float32),
        scratch_shapes=[pltpu.VMEM((_NA, 128), jnp.float32)],
        compiler_params=pltpu.CompilerParams(
            dimension_semantics=("arbitrary",),
        ),
    )(probs)


# speedup vs baseline: 2.0085x; 1.1060x over previous
import functools

import jax
import jax.numpy as jnp
from jax.experimental import pallas as pl
from jax.experimental.pallas import tpu as pltpu

_N = 1 << 24
_NA = 4
_CHUNK = _N // _NA
_GRID = 8
_BLK = _N // _GRID
_TPB = _GRID // _NA


def _segsum_softmax_body(x_ref, o_ref, acc_ref):
    i = pl.program_id(0)

    @pl.when(i == 0)
    def _init():
        acc_ref[...] = jnp.zeros_like(acc_ref)

    blk = x_ref[...].reshape(_BLK // 128, 128)
    partial = jnp.sum(blk, axis=0, keepdims=True)
    b = i // _TPB
    row = jax.lax.broadcasted_iota(jnp.int32, (_NA, 128), 0)
    acc_ref[...] += jnp.where(row == b, partial, 0.0)

    @pl.when(i == _GRID - 1)
    def _finish():
        s = jnp.sum(acc_ref[...], axis=1)
        m = jnp.max(s)
        e = jnp.exp(s - m)
        o_ref[...] = e / jnp.sum(e)


@jax.jit
def kernel(probs):
    return pl.pallas_call(
        _segsum_softmax_body,
        grid=(_GRID,),
        in_specs=[pl.BlockSpec((_BLK,), lambda i: (i,))],
        out_specs=pl.BlockSpec((_NA,), lambda i: (0,)),
        out_shape=jax.ShapeDtypeStruct((_NA,), jnp.float32),
        scratch_shapes=[pltpu.VMEM((_NA, 128), jnp.float32)],
        compiler_params=pltpu.CompilerParams(
            dimension_semantics=("arbitrary",),
        ),
    )(probs)


# speedup vs baseline: 2.0151x; 1.0032x over previous
import functools

import jax
import jax.numpy as jnp
from jax.experimental import pallas as pl
from jax.experimental.pallas import tpu as pltpu

_N = 1 << 24
_NA = 4
_CHUNK = _N // _NA
_GRID = 4
_BLK = _N // _GRID
_TPB = _GRID // _NA


def _segsum_softmax_body(x_ref, o_ref, acc_ref):
    i = pl.program_id(0)

    @pl.when(i == 0)
    def _init():
        acc_ref[...] = jnp.zeros_like(acc_ref)

    blk = x_ref[...].reshape(_BLK // 128, 128)
    partial = jnp.sum(blk, axis=0, keepdims=True)
    b = i // _TPB
    row = jax.lax.broadcasted_iota(jnp.int32, (_NA, 128), 0)
    acc_ref[...] += jnp.where(row == b, partial, 0.0)

    @pl.when(i == _GRID - 1)
    def _finish():
        s = jnp.sum(acc_ref[...], axis=1)
        m = jnp.max(s)
        e = jnp.exp(s - m)
        o_ref[...] = e / jnp.sum(e)


@jax.jit
def kernel(probs):
    return pl.pallas_call(
        _segsum_softmax_body,
        grid=(_GRID,),
        in_specs=[pl.BlockSpec((_BLK,), lambda i: (i,))],
        out_specs=pl.BlockSpec((_NA,), lambda i: (0,)),
        out_shape=jax.ShapeDtypeStruct((_NA,), jnp.float32),
        scratch_shapes=[pltpu.VMEM((_NA, 128), jnp.float32)],
        compiler_params=pltpu.CompilerParams(
            dimension_semantics=("arbitrary",),
        ),
    )(probs)
